# Initial kernel scaffold; baseline (speedup 1.0000x reference)
import jax
import jax.numpy as jnp
from jax.experimental import pallas as pl


def kernel(signal, x_grid, rec_centroid, tau, combinations):
    raise NotImplementedError("write your pallas kernel here")


# speedup vs baseline: 37.4647x; 37.4647x over previous
import functools

import numpy as np
import jax
import jax.numpy as jnp
from jax import lax
from jax.experimental import pallas as pl
from jax.experimental.pallas import tpu as pltpu
from jax.experimental.pallas import tpu_sc as plsc

_SR_MAX_TAU = 40
_LAGS = 2 * _SR_MAX_TAU + 1
_T = 4096
_KF = _T // 2 + 1
_KPAD = 17 * 128
_LPAD = 128
_NW = 32
_L = 16


def _idft_consts():
    k = np.arange(_KF)
    t = (np.arange(_LAGS) - _SR_MAX_TAU) % _T
    theta = 2.0 * np.pi * ((k[:, None] * t[None, :]) % _T) / _T
    w = np.full((_KF, 1), 2.0)
    w[0, 0] = 1.0
    w[-1, 0] = 1.0
    c = (w * np.cos(theta)) / _T
    s = (-w * np.sin(theta)) / _T
    cp = np.zeros((_KPAD, _LPAD), np.float32)
    sp = np.zeros((_KPAD, _LPAD), np.float32)
    cp[:_KF, :_LAGS] = c
    sp[:_KF, :_LAGS] = s
    return cp, sp


def _whiten_idft_body(ar_ref, ai_ref, br_ref, bi_ref, c_ref, s_ref, out_ref):
    ar = ar_ref[:, :]
    ai = ai_ref[:, :]
    br = br_ref[:, :]
    bi = bi_ref[:, :]
    gr = ar * br + ai * bi
    gi = ai * br - ar * bi
    inv = 1.0 / (jnp.sqrt(gr * gr + gi * gi) + 1e-12)
    pr = gr * inv
    pi = gi * inv
    out_ref[:, :] = (
        jnp.dot(pr, c_ref[:, :], precision=lax.Precision.HIGHEST,
                preferred_element_type=jnp.float32)
        + jnp.dot(pi, s_ref[:, :], precision=lax.Precision.HIGHEST,
                  preferred_element_type=jnp.float32)
    )


def _whiten_idft(ar, ai, br, bi, cmat, smat):
    bp = ar.shape[0]
    blk = 128
    row_spec = pl.BlockSpec((blk, _KPAD), lambda i: (i, 0))
    const_spec = pl.BlockSpec((_KPAD, _LPAD), lambda i: (0, 0))
    return pl.pallas_call(
        _whiten_idft_body,
        grid=(bp // blk,),
        in_specs=[row_spec, row_spec, row_spec, row_spec, const_spec, const_spec],
        out_specs=pl.BlockSpec((blk, _LPAD), lambda i: (i, 0)),
        out_shape=jax.ShapeDtypeStruct((bp, _LPAD), jnp.float32),
    )(ar, ai, br, bi, cmat, smat)


def _make_sc_search(n_grid, n_pairs, batch, table_cols):
    npw_raw = -(-n_grid // _NW)
    npw = -(-npw_raw // _L) * _L
    chunks = npw // _L
    n_pad = npw * _NW
    mesh = plsc.VectorSubcoreMesh(core_axis_name="c", subcore_axis_name="s")

    @functools.partial(
        pl.kernel,
        mesh=mesh,
        compiler_params=pltpu.CompilerParams(needs_layout_passes=False),
        out_type=(
            jax.ShapeDtypeStruct((_NW, batch, _L), jnp.float32),
            jax.ShapeDtypeStruct((_NW, batch, _L), jnp.int32),
        ),
        scratch_types=[
            pltpu.VMEM((batch * table_cols,), jnp.float32),
            pltpu.VMEM((n_pairs * npw,), jnp.int32),
            pltpu.VMEM((batch, _L), jnp.float32),
            pltpu.VMEM((batch, _L), jnp.int32),
        ],
    )
    def sc_search(table_hbm, fidx_hbm, omax_hbm, oidx_hbm,
                  table_v, fidx_v, rmax_v, ridx_v):
        wid = lax.axis_index("s") * 2 + lax.axis_index("c")
        pltpu.sync_copy(table_hbm, table_v)
        pltpu.sync_copy(fidx_hbm.at[wid], fidx_v)
        for b in range(batch):
            rmax_v[b, :] = jnp.full((_L,), -jnp.inf, jnp.float32)
            ridx_v[b, :] = jnp.zeros((_L,), jnp.int32)
        base = wid * npw

        def chunk_body(c, _):
            def pair_body(p, accs):
                idx = fidx_v[pl.ds(p * npw + c * _L, _L)]
                return tuple(
                    accs[b] + plsc.load_gather(table_v, [idx + b * table_cols])
                    for b in range(batch)
                )

            zeros = tuple(jnp.zeros((_L,), jnp.float32) for _ in range(batch))
            accs = lax.fori_loop(0, n_pairs, pair_body, zeros)
            nvec = base + c * _L + lax.iota(jnp.int32, _L)
            valid = nvec < n_grid
            for b in range(batch):
                m = rmax_v[b, :]
                upd = jnp.logical_and(accs[b] > m, valid)
                rmax_v[b, :] = jnp.where(upd, accs[b], m)
                ridx_v[b, :] = jnp.where(upd, nvec, ridx_v[b, :])
            return _

        lax.fori_loop(0, chunks, chunk_body, None)
        pltpu.sync_copy(rmax_v, omax_hbm.at[wid])
        pltpu.sync_copy(ridx_v, oidx_hbm.at[wid])

    return sc_search, npw, n_pad


def kernel(signal, x_grid, rec_centroid, tau, combinations):
    b_sz, m_sz, t_sz = signal.shape
    p_sz = combinations.shape[0]
    n_grid = tau.shape[0]
    table_cols = p_sz * _LAGS

    spec = jnp.fft.rfft(signal, axis=-1)
    xr = jnp.real(spec).astype(jnp.float32)
    xi = jnp.imag(spec).astype(jnp.float32)
    i0 = combinations[:, 0]
    i1 = combinations[:, 1]

    def rows(a):
        r = a.reshape(b_sz * p_sz, _KF)
        return jnp.pad(r, ((0, 0), (0, _KPAD - _KF)))

    ar = rows(xr[:, i0, :])
    ai = rows(xi[:, i0, :])
    br = rows(xr[:, i1, :])
    bi = rows(xi[:, i1, :])

    cnp, snp = _idft_consts()
    cc = _whiten_idft(ar, ai, br, bi, jnp.asarray(cnp), jnp.asarray(snp))
    table = cc[:, :_LAGS].reshape(b_sz * table_cols)

    sc_search, npw, n_pad = _make_sc_search(n_grid, p_sz, b_sz, table_cols)
    f = tau.astype(jnp.int32) + _LAGS * jnp.arange(p_sz, dtype=jnp.int32)[None, :]
    f = jnp.pad(f, ((0, n_pad - n_grid), (0, 0)))
    fidx = f.T.reshape(p_sz, _NW, npw).transpose(1, 0, 2).reshape(_NW, p_sz * npw)
    pmax, pidx = sc_search(table, fidx)

    vals = pmax.transpose(1, 0, 2).reshape(b_sz, _NW * _L)
    idxs = pidx.transpose(1, 0, 2).reshape(b_sz, _NW * _L)
    mx = vals.max(axis=1, keepdims=True)
    best = jnp.where(vals == mx, idxs, jnp.int32(2**31 - 1)).min(axis=1)
    return x_grid[best] - rec_centroid[None, :]
